# baseline (speedup 1.0000x reference)
import functools

import jax
import jax.numpy as jnp
from jax import lax
from jax.experimental import pallas as pl
from jax.experimental.pallas import tpu as pltpu
from jax.experimental.pallas import tpu_sc as plsc

N = 50000
E = 800000
C = 10
M = 32
NG = 8
F = C * NG
FH = F // 2

NSC = 2
NSUB = 16

N_PAD = 50048
ROWS_PER_TILE = N_PAD // NSUB
ZCHUNK = 46
NZ = ROWS_PER_TILE // ZCHUNK

EW = 80
EPW = E // NSUB
NB = 400
NOUTER = EPW // NB
NCHUNK = NB // EW

EWB = 125
BROWS = E // EWB
BROWS_PER_W = BROWS // (NSC * NSUB)
IBB = 8
NOUTERB = BROWS_PER_W // IBB
CZ = 136
NCZ = ROWS_PER_TILE // CZ


def _sum_body(ph_hbm, src_hbm, dst_hbm, z_hbm, out0_hbm, out1_hbm,
              srcv, dstv, dst2, rows, acc, gsem):
    c = lax.axis_index("c")
    s = lax.axis_index("s")
    row0 = s * ROWS_PER_TILE

    pltpu.sync_copy(z_hbm, rows.at[pl.ds(0, ZCHUNK), :])

    def _zero(k, _):
        pltpu.sync_copy(rows.at[pl.ds(0, ZCHUNK), :],
                        acc.at[pl.ds(row0 + k * ZCHUNK, ZCHUNK), :])
        return 0

    lax.fori_loop(0, NZ, _zero, 0)
    plsc.subcore_barrier()

    e0 = s * EPW

    def outer(ob, _):
        eb = e0 + ob * NB
        pltpu.sync_copy(src_hbm.at[pl.ds(eb, NB)], srcv)
        pltpu.sync_copy(dst_hbm.at[pl.ds(eb, NB)], dstv)
        for i in range(NB // 16):
            dst2[pl.ds(i * 16, 16)] = dstv[pl.ds(i * 16, 16)] * 2 + c
        for k in range(NCHUNK):
            pltpu.async_copy(ph_hbm.at[dst2.at[pl.ds(k * EW, EW)]],
                             rows, gsem).wait()
            pltpu.sync_copy(rows, acc.at[srcv.at[pl.ds(k * EW, EW)]],
                            add=True)
        return 0

    lax.fori_loop(0, NOUTER, outer, 0)
    plsc.subcore_barrier()

    def _wout(out_hbm):
        def _w(k, _):
            r = row0 + k * ZCHUNK
            pltpu.sync_copy(acc.at[pl.ds(r, ZCHUNK), :],
                            rows.at[pl.ds(0, ZCHUNK), :])
            pltpu.sync_copy(rows.at[pl.ds(0, ZCHUNK), :],
                            out_hbm.at[pl.ds(r, ZCHUNK), :])
            return 0
        lax.fori_loop(0, NZ, _w, 0)

    @pl.when(c == 0)
    def _():
        _wout(out0_hbm)

    @pl.when(c == 1)
    def _():
        _wout(out1_hbm)


@functools.partial(
    pl.kernel,
    out_type=(
        jax.ShapeDtypeStruct((N_PAD, FH), jnp.float32),
        jax.ShapeDtypeStruct((N_PAD, FH), jnp.float32),
    ),
    mesh=plsc.VectorSubcoreMesh(core_axis_name="c", subcore_axis_name="s"),
    compiler_params=pltpu.CompilerParams(use_tc_tiling_on_sc=False),
    scratch_types=(
        pltpu.VMEM((NB,), jnp.int32),
        pltpu.VMEM((NB,), jnp.int32),
        pltpu.VMEM((NB,), jnp.int32),
        pltpu.VMEM((EW, FH), jnp.float32),
        pltpu.VMEM_SHARED((N_PAD, FH), jnp.float32),
        pltpu.SemaphoreType.DMA,
    ),
)
def _sc_sums(*refs):
    _sum_body(*refs)


def _cnt_body(src_hbm, cz_hbm, out_hbm, srcv, onesv, cbuf, cnt):
    c = lax.axis_index("c")
    s = lax.axis_index("s")
    w = c * NSUB + s

    pltpu.sync_copy(cz_hbm, cbuf)
    row_base = s * ROWS_PER_TILE

    def _zero(k, _):
        pltpu.sync_copy(cbuf, cnt.at[pl.ds(row_base + k * CZ, CZ)])
        return 0

    lax.fori_loop(0, NCZ, _zero, 0)

    for i in range(8):
        onesv[pl.ds(i * 16, 16)] = jnp.ones((16,), jnp.float32)

    plsc.subcore_barrier()

    brow0 = w * BROWS_PER_W

    def outer(ob, _):
        pltpu.sync_copy(src_hbm.at[pl.ds(brow0 + ob * IBB, IBB), :], srcv)

        def inner(j, _):
            pltpu.sync_copy(onesv.at[pl.ds(0, EWB)],
                            cnt.at[srcv.at[j]], add=True)
            return 0

        lax.fori_loop(0, IBB, inner, 0)
        return 0

    lax.fori_loop(0, NOUTERB, outer, 0)
    plsc.subcore_barrier()

    def _w(k, _):
        r = row_base + k * CZ
        pltpu.sync_copy(cnt.at[pl.ds(r, CZ)], cbuf)
        pltpu.sync_copy(cbuf, out_hbm.at[c, pl.ds(r, CZ)])
        return 0

    lax.fori_loop(0, NCZ, _w, 0)


@functools.partial(
    pl.kernel,
    out_type=jax.ShapeDtypeStruct((NSC, N_PAD), jnp.float32),
    mesh=plsc.VectorSubcoreMesh(core_axis_name="c", subcore_axis_name="s"),
    compiler_params=pltpu.CompilerParams(use_tc_tiling_on_sc=False),
    scratch_types=(
        pltpu.VMEM((IBB, EWB), jnp.int32),
        pltpu.VMEM((128,), jnp.float32),
        pltpu.VMEM((CZ,), jnp.float32),
        pltpu.VMEM_SHARED((N_PAD,), jnp.float32),
    ),
)
def _sc_counts(*refs):
    _cnt_body(*refs)


BLK = 5000
GRID = N // BLK


def _tc_body(s0_ref, s1_ref, cnt_ref, x_ref, q_ref, b_ref, lik_ref, post_ref,
             gm_ref, smb_ref, r_ref):
    f32 = jnp.float32

    @pl.when(pl.program_id(0) == 0)
    def _():
        q = q_ref[...]
        q = q - jnp.max(q, axis=0, keepdims=True)
        eq = jnp.exp(q)
        smq = eq / jnp.sum(eq, axis=0, keepdims=True)

        b = b_ref[...]
        b = b - jnp.max(b, axis=0, keepdims=True)
        eb = jnp.exp(b)
        smb_ref[...] = eb / jnp.sum(eb, axis=0, keepdims=True)

        kt = (lax.broadcasted_iota(jnp.int32, (F, C), 1)
              == lax.broadcasted_iota(jnp.int32, (F, C), 0) // NG).astype(f32)
        g = jnp.dot(kt, smq, preferred_element_type=f32)
        gm_ref[...] = g * (
            lax.broadcasted_iota(jnp.int32, (F, F), 0) % NG
            == lax.broadcasted_iota(jnp.int32, (F, F), 1) % NG).astype(f32)

        r_ref[...] = (lax.broadcasted_iota(jnp.int32, (F, NG), 0) % NG
                      == lax.broadcasted_iota(jnp.int32, (F, NG), 1)
                      ).astype(f32)

    counts = cnt_ref[..., 0:1] + cnt_ref[..., 1:2]
    inv = 1.0 / jnp.maximum(counts, 1.0)
    aggr = jnp.concatenate([s0_ref[...], s1_ref[...]], axis=1) * inv

    t = lax.dot_general(aggr, gm_ref[...], (((1,), (1,)), ((), ())),
                        precision=lax.Precision.HIGHEST,
                        preferred_element_type=f32)

    onehot = (x_ref[...] == lax.broadcasted_iota(jnp.int32, (BLK, M), 1)
              ).astype(f32)
    bn = jnp.dot(onehot, smb_ref[...], precision=lax.Precision.HIGHEST,
                 preferred_element_type=f32)

    u = bn * t
    ssum = jnp.dot(u, r_ref[...], precision=lax.Precision.HIGHEST,
                   preferred_element_type=f32) + (C * C * 1e-8)
    sb = lax.dot_general(ssum, r_ref[...], (((1,), (1,)), ((), ())),
                         precision=lax.Precision.HIGHEST,
                         preferred_element_type=f32)
    post_ref[...] = (u + C * 1e-8) / sb
    lik_ref[...] = jnp.log(ssum)


_tc_post = pl.pallas_call(
    _tc_body,
    grid=(GRID,),
    in_specs=[
        pl.BlockSpec((BLK, FH), lambda i: (i, 0)),
        pl.BlockSpec((BLK, FH), lambda i: (i, 0)),
        pl.BlockSpec((BLK, NSC), lambda i: (i, 0)),
        pl.BlockSpec((BLK, 1), lambda i: (i, 0)),
        pl.BlockSpec((C, F), lambda i: (0, 0)),
        pl.BlockSpec((M, F), lambda i: (0, 0)),
    ],
    out_specs=[
        pl.BlockSpec((BLK, NG), lambda i: (i, 0)),
        pl.BlockSpec((BLK, F), lambda i: (i, 0)),
    ],
    out_shape=[
        jax.ShapeDtypeStruct((N, NG), jnp.float32),
        jax.ShapeDtypeStruct((N, F), jnp.float32),
    ],
    scratch_shapes=[
        pltpu.VMEM((F, F), jnp.float32),
        pltpu.VMEM((M, F), jnp.float32),
        pltpu.VMEM((F, NG), jnp.float32),
    ],
)


def kernel(x, prev_h, edge_index, Q_neigh, B):
    ph2 = prev_h.reshape(NSC * N, FH)
    src = edge_index[0].astype(jnp.int32)
    dst = edge_index[1].astype(jnp.int32)
    z = jnp.zeros((ZCHUNK, FH), jnp.float32)
    cz = jnp.zeros((CZ,), jnp.float32)

    sums0, sums1 = _sc_sums(ph2, src, dst, z)
    cnts = _sc_counts(src.reshape(BROWS, EWB), cz)

    x2d = x.astype(jnp.int32).reshape(N, 1)
    q2 = Q_neigh.reshape(C, F)
    bt = B.transpose(1, 0, 2).reshape(M, F)

    lik, post = _tc_post(sums0, sums1, cnts.T, x2d, q2, bt)
    return lik, post.reshape(N, C, NG)


# speedup vs baseline: 1.2481x; 1.2481x over previous
import functools

import jax
import jax.numpy as jnp
from jax import lax
from jax.experimental import pallas as pl
from jax.experimental.pallas import tpu as pltpu
from jax.experimental.pallas import tpu_sc as plsc

N = 50000
E = 800000
C = 10
M = 32
NG = 8
F = C * NG
FH = F // 2

NSC = 2
NSUB = 16

N_PAD = 50048
ROWS_PER_TILE = N_PAD // NSUB
ZCHUNK = 46
NZ = ROWS_PER_TILE // ZCHUNK

EW = 50
EROWS = E // EW
ROWS_PER_SUB = EROWS // NSUB
IB = 10
NOUTER = ROWS_PER_SUB // IB

EWB = 125
BROWS = E // EWB
BROWS_PER_W = BROWS // (NSC * NSUB)
IBB = 8
NOUTERB = BROWS_PER_W // IBB
CZ = 136
NCZ = ROWS_PER_TILE // CZ


def _sum_body(t0_hbm, t1_hbm, src_hbm, dst_hbm, z_hbm,
              out0_hbm, out1_hbm, srcv, dstv, rows0, rows1, acc,
              gsem0, gsem1, ssem0, ssem1):
    c = lax.axis_index("c")
    s = lax.axis_index("s")
    row0 = s * ROWS_PER_TILE

    pltpu.sync_copy(z_hbm, rows0.at[pl.ds(0, ZCHUNK), :])

    def _zero(k, _):
        pltpu.sync_copy(rows0.at[pl.ds(0, ZCHUNK), :],
                        acc.at[pl.ds(row0 + k * ZCHUNK, ZCHUNK), :])
        return 0

    lax.fori_loop(0, NZ, _zero, 0)
    plsc.subcore_barrier()

    erow0 = s * ROWS_PER_SUB
    bufs = (rows0, rows1)
    gsems = (gsem0, gsem1)
    ssems = (ssem0, ssem1)

    def _main(table_hbm):
        def outer(ob, _):
            r0 = erow0 + ob * IB
            pltpu.sync_copy(src_hbm.at[pl.ds(r0, IB), :], srcv)
            pltpu.sync_copy(dst_hbm.at[pl.ds(r0, IB), :], dstv)

            gd = [None, None]
            sd = [None, None]
            for k in range(IB):
                b = k % 2
                if sd[b] is not None:
                    sd[b].wait()
                    sd[b] = None
                gd[b] = pltpu.async_copy(table_hbm.at[dstv.at[k]],
                                         bufs[b], gsems[b])
                if k > 0:
                    pb = (k - 1) % 2
                    gd[pb].wait()
                    sd[pb] = pltpu.async_copy(bufs[pb],
                                              acc.at[srcv.at[k - 1]],
                                              ssems[pb], add=True)
            lb = (IB - 1) % 2
            gd[lb].wait()
            sd[lb] = pltpu.async_copy(bufs[lb], acc.at[srcv.at[IB - 1]],
                                      ssems[lb], add=True)
            sd[1 - lb].wait()
            sd[lb].wait()
            return 0

        lax.fori_loop(0, NOUTER, outer, 0)

    @pl.when(c == 0)
    def _():
        _main(t0_hbm)

    @pl.when(c == 1)
    def _():
        _main(t1_hbm)

    plsc.subcore_barrier()

    def _wout(out_hbm):
        def _w(k, _):
            r = row0 + k * ZCHUNK
            pltpu.sync_copy(acc.at[pl.ds(r, ZCHUNK), :],
                            rows0.at[pl.ds(0, ZCHUNK), :])
            pltpu.sync_copy(rows0.at[pl.ds(0, ZCHUNK), :],
                            out_hbm.at[pl.ds(r, ZCHUNK), :])
            return 0
        lax.fori_loop(0, NZ, _w, 0)

    @pl.when(c == 0)
    def _():
        _wout(out0_hbm)

    @pl.when(c == 1)
    def _():
        _wout(out1_hbm)


@functools.partial(
    pl.kernel,
    out_type=(
        jax.ShapeDtypeStruct((N_PAD, FH), jnp.float32),
        jax.ShapeDtypeStruct((N_PAD, FH), jnp.float32),
    ),
    mesh=plsc.VectorSubcoreMesh(core_axis_name="c", subcore_axis_name="s"),
    compiler_params=pltpu.CompilerParams(use_tc_tiling_on_sc=False),
    scratch_types=(
        pltpu.VMEM((IB, EW), jnp.int32),
        pltpu.VMEM((IB, EW), jnp.int32),
        pltpu.VMEM((EW, FH), jnp.float32),
        pltpu.VMEM((EW, FH), jnp.float32),
        pltpu.VMEM_SHARED((N_PAD, FH), jnp.float32),
        pltpu.SemaphoreType.DMA,
        pltpu.SemaphoreType.DMA,
        pltpu.SemaphoreType.DMA,
        pltpu.SemaphoreType.DMA,
    ),
)
def _sc_sums(*refs):
    _sum_body(*refs)


def _cnt_body(src_hbm, cz_hbm, out_hbm, srcv, onesv, cbuf, cnt):
    c = lax.axis_index("c")
    s = lax.axis_index("s")
    w = c * NSUB + s

    pltpu.sync_copy(cz_hbm, cbuf)
    row_base = s * ROWS_PER_TILE

    def _zero(k, _):
        pltpu.sync_copy(cbuf, cnt.at[pl.ds(row_base + k * CZ, CZ)])
        return 0

    lax.fori_loop(0, NCZ, _zero, 0)

    for i in range(8):
        onesv[pl.ds(i * 16, 16)] = jnp.ones((16,), jnp.float32)

    plsc.subcore_barrier()

    brow0 = w * BROWS_PER_W

    def outer(ob, _):
        pltpu.sync_copy(src_hbm.at[pl.ds(brow0 + ob * IBB, IBB), :], srcv)

        def inner(j, _):
            pltpu.sync_copy(onesv.at[pl.ds(0, EWB)],
                            cnt.at[srcv.at[j]], add=True)
            return 0

        lax.fori_loop(0, IBB, inner, 0)
        return 0

    lax.fori_loop(0, NOUTERB, outer, 0)
    plsc.subcore_barrier()

    def _w(k, _):
        r = row_base + k * CZ
        pltpu.sync_copy(cnt.at[pl.ds(r, CZ)], cbuf)
        pltpu.sync_copy(cbuf, out_hbm.at[c, pl.ds(r, CZ)])
        return 0

    lax.fori_loop(0, NCZ, _w, 0)


@functools.partial(
    pl.kernel,
    out_type=jax.ShapeDtypeStruct((NSC, N_PAD), jnp.float32),
    mesh=plsc.VectorSubcoreMesh(core_axis_name="c", subcore_axis_name="s"),
    compiler_params=pltpu.CompilerParams(use_tc_tiling_on_sc=False),
    scratch_types=(
        pltpu.VMEM((IBB, EWB), jnp.int32),
        pltpu.VMEM((128,), jnp.float32),
        pltpu.VMEM((CZ,), jnp.float32),
        pltpu.VMEM_SHARED((N_PAD,), jnp.float32),
    ),
)
def _sc_counts(*refs):
    _cnt_body(*refs)


BLK = 5000
GRID = N // BLK


def _tc_body(s0_ref, s1_ref, cnt_ref, x_ref, q_ref, b_ref, lik_ref, post_ref,
             gm_ref, smb_ref, r_ref):
    f32 = jnp.float32

    @pl.when(pl.program_id(0) == 0)
    def _():
        q = q_ref[...]
        q = q - jnp.max(q, axis=0, keepdims=True)
        eq = jnp.exp(q)
        smq = eq / jnp.sum(eq, axis=0, keepdims=True)

        b = b_ref[...]
        b = b - jnp.max(b, axis=0, keepdims=True)
        eb = jnp.exp(b)
        smb_ref[...] = eb / jnp.sum(eb, axis=0, keepdims=True)

        kt = (lax.broadcasted_iota(jnp.int32, (F, C), 1)
              == lax.broadcasted_iota(jnp.int32, (F, C), 0) // NG).astype(f32)
        g = jnp.dot(kt, smq, preferred_element_type=f32)
        gm_ref[...] = g * (
            lax.broadcasted_iota(jnp.int32, (F, F), 0) % NG
            == lax.broadcasted_iota(jnp.int32, (F, F), 1) % NG).astype(f32)

        r_ref[...] = (lax.broadcasted_iota(jnp.int32, (F, NG), 0) % NG
                      == lax.broadcasted_iota(jnp.int32, (F, NG), 1)
                      ).astype(f32)

    counts = cnt_ref[..., 0:1] + cnt_ref[..., 1:2]
    inv = 1.0 / jnp.maximum(counts, 1.0)
    aggr = jnp.concatenate([s0_ref[...], s1_ref[...]], axis=1) * inv

    t = lax.dot_general(aggr, gm_ref[...], (((1,), (1,)), ((), ())),
                        precision=lax.Precision.HIGHEST,
                        preferred_element_type=f32)

    onehot = (x_ref[...] == lax.broadcasted_iota(jnp.int32, (BLK, M), 1)
              ).astype(f32)
    bn = jnp.dot(onehot, smb_ref[...], precision=lax.Precision.HIGHEST,
                 preferred_element_type=f32)

    u = bn * t
    ssum = jnp.dot(u, r_ref[...], precision=lax.Precision.HIGHEST,
                   preferred_element_type=f32) + (C * C * 1e-8)
    sb = lax.dot_general(ssum, r_ref[...], (((1,), (1,)), ((), ())),
                         precision=lax.Precision.HIGHEST,
                         preferred_element_type=f32)
    post_ref[...] = (u + C * 1e-8) / sb
    lik_ref[...] = jnp.log(ssum)


_tc_post = pl.pallas_call(
    _tc_body,
    grid=(GRID,),
    in_specs=[
        pl.BlockSpec((BLK, FH), lambda i: (i, 0)),
        pl.BlockSpec((BLK, FH), lambda i: (i, 0)),
        pl.BlockSpec((BLK, NSC), lambda i: (i, 0)),
        pl.BlockSpec((BLK, 1), lambda i: (i, 0)),
        pl.BlockSpec((C, F), lambda i: (0, 0)),
        pl.BlockSpec((M, F), lambda i: (0, 0)),
    ],
    out_specs=[
        pl.BlockSpec((BLK, NG), lambda i: (i, 0)),
        pl.BlockSpec((BLK, F), lambda i: (i, 0)),
    ],
    out_shape=[
        jax.ShapeDtypeStruct((N, NG), jnp.float32),
        jax.ShapeDtypeStruct((N, F), jnp.float32),
    ],
    scratch_shapes=[
        pltpu.VMEM((F, F), jnp.float32),
        pltpu.VMEM((M, F), jnp.float32),
        pltpu.VMEM((F, NG), jnp.float32),
    ],
)


def kernel(x, prev_h, edge_index, Q_neigh, B):
    ph = prev_h.reshape(N, F)
    t0 = ph[:, :FH]
    t1 = ph[:, FH:]
    src = edge_index[0].astype(jnp.int32)
    dst = edge_index[1].astype(jnp.int32)
    z = jnp.zeros((ZCHUNK, FH), jnp.float32)
    cz = jnp.zeros((CZ,), jnp.float32)

    sums0, sums1 = _sc_sums(t0, t1, src.reshape(EROWS, EW),
                            dst.reshape(EROWS, EW), z)
    cnts = _sc_counts(src.reshape(BROWS, EWB), cz)

    x2d = x.astype(jnp.int32).reshape(N, 1)
    q2 = Q_neigh.reshape(C, F)
    bt = B.transpose(1, 0, 2).reshape(M, F)

    lik, post = _tc_post(sums0, sums1, cnts.T, x2d, q2, bt)
    return lik, post.reshape(N, C, NG)


# speedup vs baseline: 1.3635x; 1.0925x over previous
import functools

import jax
import jax.numpy as jnp
from jax import lax
from jax.experimental import pallas as pl
from jax.experimental.pallas import tpu as pltpu
from jax.experimental.pallas import tpu_sc as plsc

N = 50000
E = 800000
C = 10
M = 32
NG = 8
F = C * NG
FH = F // 2

NSC = 2
NSUB = 16

N_PAD = 50048
ROWS_PER_TILE = N_PAD // NSUB
ZCHUNK = 46
NZ = ROWS_PER_TILE // ZCHUNK

EW = 50
EROWS = E // EW
ROWS_PER_SUB = EROWS // NSUB
IB = 10
NOUTER = ROWS_PER_SUB // IB

EWB = 125
BROWS = E // EWB
BROWS_PER_W = BROWS // (NSC * NSUB)
IBB = 8
NOUTERB = BROWS_PER_W // IBB
CZ = 136
NCZ = ROWS_PER_TILE // CZ


def _sum_body(t0_hbm, t1_hbm, src_hbm, dst_hbm, z_hbm,
              out0_hbm, out1_hbm, srcv, dstv, rows0, rows1, acc,
              gsem0, gsem1, ssem0, ssem1):
    c = lax.axis_index("c")
    s = lax.axis_index("s")
    row0 = s * ROWS_PER_TILE

    pltpu.sync_copy(z_hbm, rows0.at[pl.ds(0, ZCHUNK), :])

    def _zero(k, _):
        pltpu.sync_copy(rows0.at[pl.ds(0, ZCHUNK), :],
                        acc.at[pl.ds(row0 + k * ZCHUNK, ZCHUNK), :])
        return 0

    lax.fori_loop(0, NZ, _zero, 0)
    plsc.subcore_barrier()

    erow0 = s * ROWS_PER_SUB
    bufs = (rows0, rows1)
    gsems = (gsem0, gsem1)
    ssems = (ssem0, ssem1)

    def _main(table_hbm):
        def outer(ob, _):
            r0 = erow0 + ob * IB
            pltpu.sync_copy(src_hbm.at[pl.ds(r0, IB), :], srcv)
            pltpu.sync_copy(dst_hbm.at[pl.ds(r0, IB), :], dstv)

            gd = [None, None]
            sd = [None, None]
            for k in range(IB):
                b = k % 2
                if sd[b] is not None:
                    sd[b].wait()
                    sd[b] = None
                gd[b] = pltpu.async_copy(table_hbm.at[dstv.at[k]],
                                         bufs[b], gsems[b])
                if k > 0:
                    pb = (k - 1) % 2
                    gd[pb].wait()
                    sd[pb] = pltpu.async_copy(bufs[pb],
                                              acc.at[srcv.at[k - 1]],
                                              ssems[pb], add=True)
            lb = (IB - 1) % 2
            gd[lb].wait()
            sd[lb] = pltpu.async_copy(bufs[lb], acc.at[srcv.at[IB - 1]],
                                      ssems[lb], add=True)
            sd[1 - lb].wait()
            sd[lb].wait()
            return 0

        lax.fori_loop(0, NOUTER, outer, 0)

    @pl.when(c == 0)
    def _():
        _main(t0_hbm)

    @pl.when(c == 1)
    def _():
        _main(t1_hbm)

    plsc.subcore_barrier()

    def _wout(out_hbm):
        def _w(k, _):
            r = row0 + k * ZCHUNK
            pltpu.sync_copy(acc.at[pl.ds(r, ZCHUNK), :],
                            rows0.at[pl.ds(0, ZCHUNK), :])
            pltpu.sync_copy(rows0.at[pl.ds(0, ZCHUNK), :],
                            out_hbm.at[pl.ds(r, ZCHUNK), :])
            return 0
        lax.fori_loop(0, NZ, _w, 0)

    @pl.when(c == 0)
    def _():
        _wout(out0_hbm)

    @pl.when(c == 1)
    def _():
        _wout(out1_hbm)


@functools.partial(
    pl.kernel,
    out_type=(
        jax.ShapeDtypeStruct((N_PAD, FH), jnp.float32),
        jax.ShapeDtypeStruct((N_PAD, FH), jnp.float32),
    ),
    mesh=plsc.VectorSubcoreMesh(core_axis_name="c", subcore_axis_name="s"),
    compiler_params=pltpu.CompilerParams(use_tc_tiling_on_sc=False),
    scratch_types=(
        pltpu.VMEM((IB, EW), jnp.int32),
        pltpu.VMEM((IB, EW), jnp.int32),
        pltpu.VMEM((EW, FH), jnp.float32),
        pltpu.VMEM((EW, FH), jnp.float32),
        pltpu.VMEM_SHARED((N_PAD, FH), jnp.float32),
        pltpu.SemaphoreType.DMA,
        pltpu.SemaphoreType.DMA,
        pltpu.SemaphoreType.DMA,
        pltpu.SemaphoreType.DMA,
    ),
)
def _sc_sums(*refs):
    _sum_body(*refs)


def _cnt_body(src_hbm, cz_hbm, out_hbm, srcv, onesv, cbuf, cnt):
    c = lax.axis_index("c")
    s = lax.axis_index("s")
    w = c * NSUB + s

    pltpu.sync_copy(cz_hbm, cbuf)
    row_base = s * ROWS_PER_TILE

    def _zero(k, _):
        pltpu.sync_copy(cbuf, cnt.at[pl.ds(row_base + k * CZ, CZ)])
        return 0

    lax.fori_loop(0, NCZ, _zero, 0)

    for i in range(8):
        onesv[pl.ds(i * 16, 16)] = jnp.ones((16,), jnp.float32)

    plsc.subcore_barrier()

    brow0 = w * BROWS_PER_W

    def outer(ob, _):
        pltpu.sync_copy(src_hbm.at[pl.ds(brow0 + ob * IBB, IBB), :], srcv)

        def inner(j, _):
            pltpu.sync_copy(onesv.at[pl.ds(0, EWB)],
                            cnt.at[srcv.at[j]], add=True)
            return 0

        lax.fori_loop(0, IBB, inner, 0)
        return 0

    lax.fori_loop(0, NOUTERB, outer, 0)
    plsc.subcore_barrier()

    def _w(k, _):
        r = row_base + k * CZ
        pltpu.sync_copy(cnt.at[pl.ds(r, CZ)], cbuf)
        pltpu.sync_copy(cbuf, out_hbm.at[c, pl.ds(r, CZ)])
        return 0

    lax.fori_loop(0, NCZ, _w, 0)


@functools.partial(
    pl.kernel,
    out_type=jax.ShapeDtypeStruct((NSC, N_PAD), jnp.float32),
    mesh=plsc.VectorSubcoreMesh(core_axis_name="c", subcore_axis_name="s"),
    compiler_params=pltpu.CompilerParams(use_tc_tiling_on_sc=False),
    scratch_types=(
        pltpu.VMEM((IBB, EWB), jnp.int32),
        pltpu.VMEM((128,), jnp.float32),
        pltpu.VMEM((CZ,), jnp.float32),
        pltpu.VMEM_SHARED((N_PAD,), jnp.float32),
    ),
)
def _sc_counts(*refs):
    _cnt_body(*refs)


BLK = 2000
GRID = N // BLK


def _tc_body(s0_ref, s1_ref, cnt_ref, x_ref, q_ref, b_ref, lik_ref, post_ref,
             gm_ref, smb_ref, r_ref):
    f32 = jnp.float32

    @pl.when(pl.program_id(0) == 0)
    def _():
        q = q_ref[...]
        q = q - jnp.max(q, axis=0, keepdims=True)
        eq = jnp.exp(q)
        smq = eq / jnp.sum(eq, axis=0, keepdims=True)

        b = b_ref[...]
        b = b - jnp.max(b, axis=0, keepdims=True)
        eb = jnp.exp(b)
        smb_ref[...] = eb / jnp.sum(eb, axis=0, keepdims=True)

        kt = (lax.broadcasted_iota(jnp.int32, (F, C), 1)
              == lax.broadcasted_iota(jnp.int32, (F, C), 0) // NG).astype(f32)
        g = jnp.dot(kt, smq, preferred_element_type=f32)
        gm_ref[...] = g * (
            lax.broadcasted_iota(jnp.int32, (F, F), 0) % NG
            == lax.broadcasted_iota(jnp.int32, (F, F), 1) % NG).astype(f32)

        r_ref[...] = (lax.broadcasted_iota(jnp.int32, (F, NG), 0) % NG
                      == lax.broadcasted_iota(jnp.int32, (F, NG), 1)
                      ).astype(f32)

    counts = cnt_ref[..., 0:1] + cnt_ref[..., 1:2]
    inv = 1.0 / jnp.maximum(counts, 1.0)
    aggr = jnp.concatenate([s0_ref[...], s1_ref[...]], axis=1) * inv

    t = lax.dot_general(aggr, gm_ref[...], (((1,), (1,)), ((), ())),
                        precision=lax.Precision.HIGHEST,
                        preferred_element_type=f32)

    onehot = (x_ref[...] == lax.broadcasted_iota(jnp.int32, (BLK, M), 1)
              ).astype(f32)
    bn = jnp.dot(onehot, smb_ref[...], precision=lax.Precision.HIGHEST,
                 preferred_element_type=f32)

    u = bn * t
    ssum = jnp.dot(u, r_ref[...], precision=lax.Precision.HIGHEST,
                   preferred_element_type=f32) + (C * C * 1e-8)
    sb = lax.dot_general(ssum, r_ref[...], (((1,), (1,)), ((), ())),
                         precision=lax.Precision.HIGHEST,
                         preferred_element_type=f32)
    post_ref[...] = (u + C * 1e-8) / sb
    lik_ref[...] = jnp.log(ssum)


_tc_post = pl.pallas_call(
    _tc_body,
    grid=(GRID,),
    in_specs=[
        pl.BlockSpec((BLK, FH), lambda i: (i, 0)),
        pl.BlockSpec((BLK, FH), lambda i: (i, 0)),
        pl.BlockSpec((BLK, NSC), lambda i: (i, 0)),
        pl.BlockSpec((BLK, 1), lambda i: (i, 0)),
        pl.BlockSpec((C, F), lambda i: (0, 0)),
        pl.BlockSpec((M, F), lambda i: (0, 0)),
    ],
    out_specs=[
        pl.BlockSpec((BLK, NG), lambda i: (i, 0)),
        pl.BlockSpec((BLK, F), lambda i: (i, 0)),
    ],
    out_shape=[
        jax.ShapeDtypeStruct((N, NG), jnp.float32),
        jax.ShapeDtypeStruct((N, F), jnp.float32),
    ],
    scratch_shapes=[
        pltpu.VMEM((F, F), jnp.float32),
        pltpu.VMEM((M, F), jnp.float32),
        pltpu.VMEM((F, NG), jnp.float32),
    ],
)


def kernel(x, prev_h, edge_index, Q_neigh, B):
    ph = prev_h.reshape(N, F)
    t0 = ph[:, :FH]
    t1 = ph[:, FH:]
    src = edge_index[0].astype(jnp.int32)
    dst = edge_index[1].astype(jnp.int32)
    z = jnp.zeros((ZCHUNK, FH), jnp.float32)
    cz = jnp.zeros((CZ,), jnp.float32)

    sums0, sums1 = _sc_sums(t0, t1, src.reshape(EROWS, EW),
                            dst.reshape(EROWS, EW), z)
    cnts = _sc_counts(src.reshape(BROWS, EWB), cz)

    x2d = x.astype(jnp.int32).reshape(N, 1)
    q2 = Q_neigh.reshape(C, F)
    bt = B.transpose(1, 0, 2).reshape(M, F)

    lik, post = _tc_post(sums0, sums1, cnts.T, x2d, q2, bt)
    return lik, post.reshape(N, C, NG)


# speedup vs baseline: 1.3760x; 1.0092x over previous
import functools

import jax
import jax.numpy as jnp
from jax import lax
from jax.experimental import pallas as pl
from jax.experimental.pallas import tpu as pltpu
from jax.experimental.pallas import tpu_sc as plsc

N = 50000
E = 800000
C = 10
M = 32
NG = 8
F = C * NG
FH = F // 2

NSC = 2
NSUB = 16

N_PAD = 50048
ROWS_PER_TILE = N_PAD // NSUB
ZCHUNK = 46
NZ = ROWS_PER_TILE // ZCHUNK

EW = 50
EROWS = E // EW
ROWS_PER_SUB = EROWS // NSUB
IB = 10
NOUTER = ROWS_PER_SUB // IB

EWB = 125
BROWS = E // EWB
BROWS_PER_W = BROWS // (NSC * NSUB)
IBB = 8
NOUTERB = BROWS_PER_W // IBB
CZ = 136
NCZ = ROWS_PER_TILE // CZ


def _sum_body(t0_hbm, t1_hbm, src_hbm, dst_hbm, z_hbm,
              out0_hbm, out1_hbm, srcv, dstv, rows0, rows1, acc,
              gsem0, gsem1, ssem0, ssem1):
    c = lax.axis_index("c")
    s = lax.axis_index("s")
    row0 = s * ROWS_PER_TILE

    pltpu.sync_copy(z_hbm, rows0.at[pl.ds(0, ZCHUNK), :])

    def _zero(k, _):
        pltpu.sync_copy(rows0.at[pl.ds(0, ZCHUNK), :],
                        acc.at[pl.ds(row0 + k * ZCHUNK, ZCHUNK), :])
        return 0

    lax.fori_loop(0, NZ, _zero, 0)
    plsc.subcore_barrier()

    erow0 = s * ROWS_PER_SUB
    bufs = (rows0, rows1)
    gsems = (gsem0, gsem1)
    ssems = (ssem0, ssem1)

    def _main(table_hbm):
        def outer(ob, _):
            r0 = erow0 + ob * IB
            pltpu.sync_copy(src_hbm.at[pl.ds(r0, IB), :], srcv)
            pltpu.sync_copy(dst_hbm.at[pl.ds(r0, IB), :], dstv)

            gd = [None, None]
            sd = [None, None]
            for k in range(IB):
                b = k % 2
                if sd[b] is not None:
                    sd[b].wait()
                    sd[b] = None
                gd[b] = pltpu.async_copy(table_hbm.at[dstv.at[k]],
                                         bufs[b], gsems[b])
                if k > 0:
                    pb = (k - 1) % 2
                    gd[pb].wait()
                    sd[pb] = pltpu.async_copy(bufs[pb],
                                              acc.at[srcv.at[k - 1]],
                                              ssems[pb], add=True)
            lb = (IB - 1) % 2
            gd[lb].wait()
            sd[lb] = pltpu.async_copy(bufs[lb], acc.at[srcv.at[IB - 1]],
                                      ssems[lb], add=True)
            sd[1 - lb].wait()
            sd[lb].wait()
            return 0

        lax.fori_loop(0, NOUTER, outer, 0)

    @pl.when(c == 0)
    def _():
        _main(t0_hbm)

    @pl.when(c == 1)
    def _():
        _main(t1_hbm)

    plsc.subcore_barrier()

    def _wout(out_hbm):
        def _w(k, _):
            r = row0 + k * ZCHUNK
            pltpu.sync_copy(acc.at[pl.ds(r, ZCHUNK), :],
                            rows0.at[pl.ds(0, ZCHUNK), :])
            pltpu.sync_copy(rows0.at[pl.ds(0, ZCHUNK), :],
                            out_hbm.at[pl.ds(r, ZCHUNK), :])
            return 0
        lax.fori_loop(0, NZ, _w, 0)

    @pl.when(c == 0)
    def _():
        _wout(out0_hbm)

    @pl.when(c == 1)
    def _():
        _wout(out1_hbm)


@functools.partial(
    pl.kernel,
    out_type=(
        jax.ShapeDtypeStruct((N_PAD, FH), jnp.float32),
        jax.ShapeDtypeStruct((N_PAD, FH), jnp.float32),
    ),
    mesh=plsc.VectorSubcoreMesh(core_axis_name="c", subcore_axis_name="s"),
    compiler_params=pltpu.CompilerParams(use_tc_tiling_on_sc=False),
    scratch_types=(
        pltpu.VMEM((IB, EW), jnp.int32),
        pltpu.VMEM((IB, EW), jnp.int32),
        pltpu.VMEM((EW, FH), jnp.float32),
        pltpu.VMEM((EW, FH), jnp.float32),
        pltpu.VMEM_SHARED((N_PAD, FH), jnp.float32),
        pltpu.SemaphoreType.DMA,
        pltpu.SemaphoreType.DMA,
        pltpu.SemaphoreType.DMA,
        pltpu.SemaphoreType.DMA,
    ),
)
def _sc_sums(*refs):
    _sum_body(*refs)


TCH = 125
TROWS = N // NSUB
NTCH = TROWS // TCH


def _cnt_body(src_hbm, ph_hbm, cz_hbm, out_hbm, t0_hbm, t1_hbm,
              srcv, onesv, cbuf, tbuf, cnt):
    c = lax.axis_index("c")
    s = lax.axis_index("s")
    w = c * NSUB + s

    def _split(t_hbm, co):
        def _t(k, _):
            r = s * TROWS + k * TCH
            pltpu.sync_copy(ph_hbm.at[pl.ds(r, TCH), pl.ds(co, FH)], tbuf)
            pltpu.sync_copy(tbuf, t_hbm.at[pl.ds(r, TCH), :])
            return 0
        lax.fori_loop(0, NTCH, _t, 0)

    @pl.when(c == 0)
    def _():
        _split(t0_hbm, 0)

    @pl.when(c == 1)
    def _():
        _split(t1_hbm, FH)

    pltpu.sync_copy(cz_hbm, cbuf)
    row_base = s * ROWS_PER_TILE

    def _zero(k, _):
        pltpu.sync_copy(cbuf, cnt.at[pl.ds(row_base + k * CZ, CZ)])
        return 0

    lax.fori_loop(0, NCZ, _zero, 0)

    for i in range(8):
        onesv[pl.ds(i * 16, 16)] = jnp.ones((16,), jnp.float32)

    plsc.subcore_barrier()

    brow0 = w * BROWS_PER_W

    def outer(ob, _):
        pltpu.sync_copy(src_hbm.at[pl.ds(brow0 + ob * IBB, IBB), :], srcv)

        def inner(j, _):
            pltpu.sync_copy(onesv.at[pl.ds(0, EWB)],
                            cnt.at[srcv.at[j]], add=True)
            return 0

        lax.fori_loop(0, IBB, inner, 0)
        return 0

    lax.fori_loop(0, NOUTERB, outer, 0)
    plsc.subcore_barrier()

    def _w(k, _):
        r = row_base + k * CZ
        pltpu.sync_copy(cnt.at[pl.ds(r, CZ)], cbuf)
        pltpu.sync_copy(cbuf, out_hbm.at[c, pl.ds(r, CZ)])
        return 0

    lax.fori_loop(0, NCZ, _w, 0)


@functools.partial(
    pl.kernel,
    out_type=(
        jax.ShapeDtypeStruct((NSC, N_PAD), jnp.float32),
        jax.ShapeDtypeStruct((N, FH), jnp.float32),
        jax.ShapeDtypeStruct((N, FH), jnp.float32),
    ),
    mesh=plsc.VectorSubcoreMesh(core_axis_name="c", subcore_axis_name="s"),
    compiler_params=pltpu.CompilerParams(use_tc_tiling_on_sc=False),
    scratch_types=(
        pltpu.VMEM((IBB, EWB), jnp.int32),
        pltpu.VMEM((128,), jnp.float32),
        pltpu.VMEM((CZ,), jnp.float32),
        pltpu.VMEM((TCH, FH), jnp.float32),
        pltpu.VMEM_SHARED((N_PAD,), jnp.float32),
    ),
)
def _sc_counts(*refs):
    _cnt_body(*refs)


BLK = 2000
GRID = N // BLK


def _tc_body(s0_ref, s1_ref, cnt_ref, x_ref, q_ref, b_ref, lik_ref, post_ref,
             gm_ref, smb_ref, r_ref):
    f32 = jnp.float32

    @pl.when(pl.program_id(0) == 0)
    def _():
        q = q_ref[...]
        q = q - jnp.max(q, axis=0, keepdims=True)
        eq = jnp.exp(q)
        smq = eq / jnp.sum(eq, axis=0, keepdims=True)

        b = b_ref[...]
        b = b - jnp.max(b, axis=0, keepdims=True)
        eb = jnp.exp(b)
        smb_ref[...] = eb / jnp.sum(eb, axis=0, keepdims=True)

        kt = (lax.broadcasted_iota(jnp.int32, (F, C), 1)
              == lax.broadcasted_iota(jnp.int32, (F, C), 0) // NG).astype(f32)
        g = jnp.dot(kt, smq, preferred_element_type=f32)
        gm_ref[...] = g * (
            lax.broadcasted_iota(jnp.int32, (F, F), 0) % NG
            == lax.broadcasted_iota(jnp.int32, (F, F), 1) % NG).astype(f32)

        r_ref[...] = (lax.broadcasted_iota(jnp.int32, (F, NG), 0) % NG
                      == lax.broadcasted_iota(jnp.int32, (F, NG), 1)
                      ).astype(f32)

    counts = cnt_ref[..., 0:1] + cnt_ref[..., 1:2]
    inv = 1.0 / jnp.maximum(counts, 1.0)
    aggr = jnp.concatenate([s0_ref[...], s1_ref[...]], axis=1) * inv

    t = lax.dot_general(aggr, gm_ref[...], (((1,), (1,)), ((), ())),
                        precision=lax.Precision.HIGHEST,
                        preferred_element_type=f32)

    onehot = (x_ref[...] == lax.broadcasted_iota(jnp.int32, (BLK, M), 1)
              ).astype(f32)
    bn = jnp.dot(onehot, smb_ref[...], precision=lax.Precision.HIGHEST,
                 preferred_element_type=f32)

    u = bn * t
    ssum = jnp.dot(u, r_ref[...], precision=lax.Precision.HIGHEST,
                   preferred_element_type=f32) + (C * C * 1e-8)
    sb = lax.dot_general(ssum, r_ref[...], (((1,), (1,)), ((), ())),
                         precision=lax.Precision.HIGHEST,
                         preferred_element_type=f32)
    post_ref[...] = (u + C * 1e-8) / sb
    lik_ref[...] = jnp.log(ssum)


_tc_post = pl.pallas_call(
    _tc_body,
    grid=(GRID,),
    in_specs=[
        pl.BlockSpec((BLK, FH), lambda i: (i, 0)),
        pl.BlockSpec((BLK, FH), lambda i: (i, 0)),
        pl.BlockSpec((BLK, NSC), lambda i: (i, 0)),
        pl.BlockSpec((BLK, 1), lambda i: (i, 0)),
        pl.BlockSpec((C, F), lambda i: (0, 0)),
        pl.BlockSpec((M, F), lambda i: (0, 0)),
    ],
    out_specs=[
        pl.BlockSpec((BLK, NG), lambda i: (i, 0)),
        pl.BlockSpec((BLK, F), lambda i: (i, 0)),
    ],
    out_shape=[
        jax.ShapeDtypeStruct((N, NG), jnp.float32),
        jax.ShapeDtypeStruct((N, F), jnp.float32),
    ],
    scratch_shapes=[
        pltpu.VMEM((F, F), jnp.float32),
        pltpu.VMEM((M, F), jnp.float32),
        pltpu.VMEM((F, NG), jnp.float32),
    ],
)


def kernel(x, prev_h, edge_index, Q_neigh, B):
    ph = prev_h.reshape(N, F)
    src = edge_index[0].astype(jnp.int32)
    dst = edge_index[1].astype(jnp.int32)
    z = jnp.zeros((ZCHUNK, FH), jnp.float32)
    cz = jnp.zeros((CZ,), jnp.float32)

    cnts, t0, t1 = _sc_counts(src.reshape(BROWS, EWB), ph, cz)
    sums0, sums1 = _sc_sums(t0, t1, src.reshape(EROWS, EW),
                            dst.reshape(EROWS, EW), z)

    x2d = x.astype(jnp.int32).reshape(N, 1)
    q2 = Q_neigh.reshape(C, F)
    bt = B.transpose(1, 0, 2).reshape(M, F)

    lik, post = _tc_post(sums0, sums1, cnts.T, x2d, q2, bt)
    return lik, post.reshape(N, C, NG)
